# baseline (speedup 1.0000x reference)
import functools

import jax
import jax.numpy as jnp
from jax import lax
from jax.experimental import pallas as pl
from jax.experimental.pallas import tpu as pltpu
from jax.experimental.pallas import tpu_sc as plsc

TOPK = 32
_ROW_BLK = 512
_PAIR_BLK = 512

_NC = 2
_NS = 16
_NW = _NC * _NS
_GCHUNK = 128


def _topk_agg_body(sim_ref, feat_ref, w_ref, b_ref, out_ref):
  blk, n = sim_ref.shape[1], sim_ref.shape[2]
  j = pl.program_id(1)
  s = sim_ref[0]
  col = lax.broadcasted_iota(jnp.int32, (blk, n), 1)
  row = lax.broadcasted_iota(jnp.int32, (blk, n), 0) + j * blk
  s = jnp.where(col == row, 0.0, s)

  nch = n // 128
  m1 = s[:, :128]
  for c in range(1, nch):
    m1 = jnp.maximum(m1, s[:, c * 128:(c + 1) * 128])
  m2 = jnp.maximum(jnp.maximum(m1[:, :32], m1[:, 32:64]),
                   jnp.maximum(m1[:, 64:96], m1[:, 96:128]))
  lo0 = lax.bitcast_convert_type(jnp.min(m2, axis=1, keepdims=True),
                                 jnp.int32)
  hi0 = lax.bitcast_convert_type(jnp.max(m2, axis=1, keepdims=True),
                                 jnp.int32)

  def bis_cond(carry):
    i, lo, hi, _, _ = carry
    return jnp.logical_and(i < 64, jnp.any(lo < hi))

  def probe(lo, hi, cl, ch, mid):
    mid_f = lax.bitcast_convert_type(mid, jnp.float32)
    cnt = jnp.sum((s >= mid_f).astype(jnp.float32), axis=1, keepdims=True)
    ge = cnt >= TOPK
    return (jnp.where(ge, mid, lo), jnp.where(ge, hi, mid - 1),
            jnp.where(ge, cnt, cl), jnp.where(ge, ch, cnt))

  def bis(carry):
    i, lo, hi, cl, ch = carry
    lo_f = lax.bitcast_convert_type(lo, jnp.float32)
    hi_f = lax.bitcast_convert_type(hi, jnp.float32)
    frac = (cl - (TOPK - 0.5)) / jnp.maximum(cl - ch, 1.0)
    est = lo_f + (hi_f - lo_f) * frac
    mid = jnp.clip(lax.bitcast_convert_type(est, jnp.int32), lo + 1, hi)
    lo, hi, cl, ch = probe(lo, hi, cl, ch, mid)
    mid = lo + ((hi - lo + 1) >> 1)
    mid = jnp.minimum(mid, hi)
    lo, hi, cl, ch = probe(lo, hi, cl, ch, mid)
    return i + 2, lo, hi, cl, ch

  cl0 = jnp.full((blk, 1), float(n), jnp.float32)
  ch0 = jnp.zeros((blk, 1), jnp.float32)
  _, t, _, _, _ = lax.while_loop(bis_cond, bis,
                                 (jnp.int32(0), lo0, hi0, cl0, ch0))

  t_f = lax.bitcast_convert_type(t, jnp.float32)
  gt = s > t_f
  eq = s == t_f
  cnt_gt = jnp.sum(gt.astype(jnp.int32), axis=1, keepdims=True)
  cnt_ge = cnt_gt + jnp.sum(eq.astype(jnp.int32), axis=1, keepdims=True)
  m = TOPK - cnt_gt

  def no_tie():
    return jnp.full((blk, 1), n - 1, jnp.int32)

  def with_tie():
    jlo0 = jnp.full((blk, 1), -1, jnp.int32)
    jhi0 = jnp.full((blk, 1), n - 1, jnp.int32)

    def jb(_, carry):
      jlo, jhi = carry
      mid = (jlo + jhi) >> 1
      cnt = jnp.sum((eq & (col <= mid)).astype(jnp.int32), axis=1,
                    keepdims=True)
      ok = cnt >= m
      return jnp.where(ok, jlo, mid + 1), jnp.where(ok, mid, jhi)

    _, jstar = lax.fori_loop(0, 13, jb, (jlo0, jhi0))
    return jstar

  jstar = lax.cond(jnp.any(cnt_ge != TOPK), with_tie, no_tie)
  mask = gt | (eq & (col <= jstar))

  w = jnp.where(mask, s, 0.0)
  ssum = jnp.maximum(jnp.sum(w, axis=1, keepdims=True), 1e-8)
  token = jnp.dot(w.astype(jnp.bfloat16), feat_ref[...],
                  preferred_element_type=jnp.float32) / ssum
  proj = lax.dot_general(token, w_ref[0], (((1,), (1,)), ((), ())),
                         preferred_element_type=jnp.float32)
  out_ref[0] = jnp.maximum(proj + b_ref[0], 0.0)


def _topk_agg_call(sim, feat, proj_w, proj_b):
  v, n, _ = sim.shape
  d = feat.shape[1]
  blk = min(_ROW_BLK, n)
  grid = (v, n // blk)
  return pl.pallas_call(
      _topk_agg_body,
      grid=grid,
      in_specs=[
          pl.BlockSpec((1, blk, n), lambda i, j: (i, j, 0)),
          pl.BlockSpec((n, d), lambda i, j: (0, 0)),
          pl.BlockSpec((1, d, d), lambda i, j: (i, 0, 0)),
          pl.BlockSpec((1, 1, d), lambda i, j: (i, 0, 0)),
      ],
      out_specs=pl.BlockSpec((1, blk, d), lambda i, j: (i, j, 0)),
      out_shape=jax.ShapeDtypeStruct((v, n, d), jnp.float32),
      compiler_params=pltpu.CompilerParams(
          dimension_semantics=("arbitrary", "arbitrary")),
  )(sim, feat, proj_w, proj_b.reshape(v, 1, d))


def _gather_rows(table, idx):
  nrows, d = table.shape
  b = idx.shape[0]
  per_w = b // _NW
  nch = per_w // _GCHUNK
  mesh = plsc.VectorSubcoreMesh(core_axis_name="c", subcore_axis_name="s",
                                num_cores=_NC, num_subcores=_NS)

  @functools.partial(
      pl.kernel,
      mesh=mesh,
      out_type=jax.ShapeDtypeStruct((b, d), jnp.float32),
      scratch_types=[
          pltpu.VMEM((_GCHUNK,), jnp.int32),
          pltpu.VMEM((_GCHUNK, d), jnp.float32),
          pltpu.SemaphoreType.DMA,
      ],
  )
  def k(idx_hbm, table_hbm, out_hbm, idx_v, rows_v, sem):
    wid = lax.axis_index("s") * _NC + lax.axis_index("c")
    base = wid * per_w
    for c in range(nch):
      off = base + c * _GCHUNK
      pltpu.sync_copy(idx_hbm.at[pl.ds(off, _GCHUNK)], idx_v)
      pltpu.async_copy(table_hbm.at[idx_v], rows_v, sem).wait()
      pltpu.sync_copy(rows_v, out_hbm.at[pl.ds(off, _GCHUNK)])

  return k(idx, table)


def _att_fusion(tok, emb, wq, bq, wk, bk, wv, bv, wo, bo, d, norm):
  t0 = tok[:, :d]
  t1 = tok[:, d:2 * d]
  t2 = tok[:, 2 * d:]

  def lin(x, wmat, bvec):
    return lax.dot_general(x, wmat, (((1,), (1,)), ((), ())),
                           preferred_element_type=jnp.float32) + bvec

  q = lin(emb, wq, bq)
  s0 = jnp.sum(q * lin(t0, wk, bk), axis=1, keepdims=True) * norm
  s1 = jnp.sum(q * lin(t1, wk, bk), axis=1, keepdims=True) * norm
  s2 = jnp.sum(q * lin(t2, wk, bk), axis=1, keepdims=True) * norm
  mx = jnp.maximum(jnp.maximum(s0, s1), s2)
  e0 = jnp.exp(s0 - mx)
  e1 = jnp.exp(s1 - mx)
  e2 = jnp.exp(s2 - mx)
  z = e0 + e1 + e2
  fused = (e0 * lin(t0, wv, bv) + e1 * lin(t1, wv, bv)
           + e2 * lin(t2, wv, bv)) / z
  mean = (t0 + t1 + t2) * (1.0 / 3.0)
  var = ((t0 - mean) ** 2 + (t1 - mean) ** 2 + (t2 - mean) ** 2) * 0.5
  dis = jnp.sqrt(var)
  cat = jnp.concatenate((fused, dis), axis=1)
  return jnp.maximum(lin(cat, wo, bo), 0.0)


def _fusion_mlp_body(mtok_ref, dtok_ref, memb_ref, demb_ref, conf_ref,
                     mWq, mbq, mWk, mbk, mWv, mbv, mWo, mbo,
                     dWq, dbq, dWk, dbk, dWv, dbv, dWo, dbo,
                     W1a, w1c, b1, W2, b2, out_ref):
  d = memb_ref.shape[1]
  norm = 1.0 / (d ** 0.5)
  mi_emb = memb_ref[...]
  dj_emb = demb_ref[...]
  mi_sem = _att_fusion(mtok_ref[...], mi_emb, mWq[...], mbq[...], mWk[...],
                       mbk[...], mWv[...], mbv[...], mWo[...], mbo[...], d,
                       norm)
  dj_sem = _att_fusion(dtok_ref[...], dj_emb, dWq[...], dbq[...], dWk[...],
                       dbk[...], dWv[...], dbv[...], dWo[...], dbo[...], d,
                       norm)
  big = jnp.concatenate(
      (mi_emb, dj_emb, mi_sem, dj_sem, jnp.abs(mi_sem - dj_sem),
       mi_sem * dj_sem), axis=1)
  h = lax.dot_general(big, W1a[...], (((1,), (1,)), ((), ())),
                      preferred_element_type=jnp.float32)
  h = jnp.maximum(h + conf_ref[...] * w1c[...] + b1[...], 0.0)
  out = lax.dot_general(h, W2[...], (((1,), (1,)), ((), ())),
                        preferred_element_type=jnp.float32)
  out_ref[...] = jnp.maximum(out + b2[...], 0.0)


def _fusion_mlp_call(m_tok, d_tok, mi_emb, dj_emb, conf, mf, df, W1a, w1c,
                     b1, W2, b2):
  b, d = mi_emb.shape
  hid = W2.shape[0]
  blk = min(_PAIR_BLK, b)
  grid = (b // blk,)

  def rowblk(shape):
    return pl.BlockSpec((blk,) + shape[1:], lambda i: (i,) + (0,) * (len(shape) - 1))

  def whole(shape):
    return pl.BlockSpec(shape, lambda i: (0,) * len(shape))

  fspecs = []
  fargs = []
  for side in (mf, df):
    for arr in side:
      a = arr if arr.ndim == 2 else arr.reshape(1, -1)
      fargs.append(a)
      fspecs.append(whole(a.shape))

  args = [m_tok, d_tok, mi_emb, dj_emb, conf.reshape(b, 1)] + fargs + [
      W1a, w1c.reshape(1, hid), b1.reshape(1, hid), W2, b2.reshape(1, hid)]
  specs = [rowblk(m_tok.shape), rowblk(d_tok.shape), rowblk(mi_emb.shape),
           rowblk(dj_emb.shape), rowblk((b, 1))] + fspecs + [
      whole(W1a.shape), whole((1, hid)), whole((1, hid)), whole(W2.shape),
      whole((1, hid))]

  return pl.pallas_call(
      _fusion_mlp_body,
      grid=grid,
      in_specs=specs,
      out_specs=rowblk((b, hid)),
      out_shape=jax.ShapeDtypeStruct((b, hid), jnp.float32),
      compiler_params=pltpu.CompilerParams(
          dimension_semantics=("arbitrary",)),
  )(*args)


def kernel(sim_mm, sim_dd, all_node_feat, m_node, d_node, mi_emb, dj_emb,
           pair_confidence, m_proj_W, m_proj_b, d_proj_W, d_proj_b,
           mf_Wq, mf_bq, mf_Wk, mf_bk, mf_Wv, mf_bv, mf_Wo, mf_bo,
           df_Wq, df_bq, df_Wk, df_bk, df_Wv, df_bv, df_Wo, df_bo,
           pm_W1, pm_b1, pm_W2, pm_b2):
  v, n_m, _ = sim_mm.shape
  n_d = sim_dd.shape[1]
  d = all_node_feat.shape[1]
  b = m_node.shape[0]

  mi_feat = all_node_feat[:n_m]
  di_feat = all_node_feat[n_m:]

  proj_m = _topk_agg_call(sim_mm, mi_feat.astype(jnp.bfloat16), m_proj_W,
                          m_proj_b)
  proj_d = _topk_agg_call(sim_dd, di_feat.astype(jnp.bfloat16), d_proj_W,
                          d_proj_b)

  offs_m = jnp.arange(v, dtype=jnp.int32) * n_m
  offs_d = jnp.arange(v, dtype=jnp.int32) * n_d
  idx_m = (m_node.astype(jnp.int32)[:, None] + offs_m[None, :]).reshape(-1)
  idx_d = (d_node.astype(jnp.int32)[:, None] + offs_d[None, :]).reshape(-1)

  m_tok = _gather_rows(proj_m.reshape(v * n_m, d), idx_m).reshape(b, v * d)
  d_tok = _gather_rows(proj_d.reshape(v * n_d, d), idx_d).reshape(b, v * d)

  mf = (mf_Wq, mf_bq, mf_Wk, mf_bk, mf_Wv, mf_bv, mf_Wo, mf_bo)
  df = (df_Wq, df_bq, df_Wk, df_bk, df_Wv, df_bv, df_Wo, df_bo)
  d_in = pm_W1.shape[1] - 1
  return _fusion_mlp_call(m_tok, d_tok, mi_emb, dj_emb, pair_confidence,
                          mf, df, pm_W1[:, :d_in], pm_W1[:, d_in],
                          pm_b1, pm_W2, pm_b2)


# speedup vs baseline: 1.1649x; 1.1649x over previous
import functools

import jax
import jax.numpy as jnp
from jax import lax
from jax.experimental import pallas as pl
from jax.experimental.pallas import tpu as pltpu
from jax.experimental.pallas import tpu_sc as plsc

TOPK = 32
_ROW_BLK = 512
_PAIR_BLK = 512

_NC = 2
_NS = 16
_NW = _NC * _NS
_GCHUNK = 128


def _topk_agg_body(sim_ref, feat_ref, w_ref, b_ref, out_ref):
  blk, n = sim_ref.shape[1], sim_ref.shape[2]
  j = pl.program_id(1)
  s = sim_ref[0]
  col = lax.broadcasted_iota(jnp.int32, (blk, n), 1)
  row = lax.broadcasted_iota(jnp.int32, (blk, n), 0) + j * blk
  s = jnp.where(col == row, 0.0, s)

  nch = n // 128
  m1 = s[:, :128]
  for c in range(1, nch):
    m1 = jnp.maximum(m1, s[:, c * 128:(c + 1) * 128])
  m2 = jnp.maximum(jnp.maximum(m1[:, :32], m1[:, 32:64]),
                   jnp.maximum(m1[:, 64:96], m1[:, 96:128]))
  lo0 = lax.bitcast_convert_type(jnp.min(m2, axis=1, keepdims=True),
                                 jnp.int32)
  hi0 = lax.bitcast_convert_type(jnp.max(m2, axis=1, keepdims=True),
                                 jnp.int32)

  p1 = jnp.maximum(s[:, :n // 2], s[:, n // 2:])
  q = jnp.max(jnp.minimum(s[:, :n // 2], s[:, n // 2:]), axis=1,
              keepdims=True)
  p2 = jnp.maximum(p1[:, :n // 4], p1[:, n // 4:])
  q = jnp.maximum(
      q, jnp.max(jnp.minimum(p1[:, :n // 4], p1[:, n // 4:]), axis=1,
                 keepdims=True))
  p3 = jnp.maximum(p2[:, :n // 8], p2[:, n // 8:])
  q = jnp.maximum(
      q, jnp.max(jnp.minimum(p2[:, :n // 8], p2[:, n // 8:]), axis=1,
                 keepdims=True))
  qb = lax.bitcast_convert_type(q, jnp.int32)

  def make_step(data):
    def step(lo, hi):
      mid = lo + ((hi - lo + 1) >> 1)
      mid_f = lax.bitcast_convert_type(mid, jnp.float32)
      cnt = jnp.sum((data >= mid_f).astype(jnp.int32), axis=1, keepdims=True)
      ge = cnt >= TOPK
      return jnp.where(ge, mid, lo), jnp.where(ge, hi, mid - 1)
    return step

  full_step = make_step(s)
  fold_step = make_step(p3)

  def p1_cond(carry):
    i, lo, hi = carry
    open_ = lo < hi
    return jnp.logical_and(i < 32,
                           jnp.any(jnp.logical_and(open_, lo < qb)))

  def p1_body(carry):
    i, lo, hi = carry
    for _ in range(2):
      lo, hi = full_step(lo, hi)
    return i + 2, lo, hi

  def p2_cond(carry):
    i, lo, hi = carry
    return jnp.logical_and(i < 34, jnp.any(lo < hi))

  def p2_body(carry):
    i, lo, hi = carry
    for _ in range(4):
      lo, hi = fold_step(lo, hi)
    return i + 4, lo, hi

  st = lax.while_loop(p1_cond, p1_body, (jnp.int32(0), lo0, hi0))
  _, t, _ = lax.while_loop(p2_cond, p2_body, st)

  t_f = lax.bitcast_convert_type(t, jnp.float32)
  ge = s >= t_f
  cnt_ge = jnp.sum(ge.astype(jnp.int32), axis=1, keepdims=True)

  def no_tie():
    return jnp.full((blk, 1), n - 1, jnp.int32)

  def with_tie():
    eq = ge & (s == t_f)
    cnt_gt = cnt_ge - jnp.sum(eq.astype(jnp.int32), axis=1, keepdims=True)
    m = TOPK - cnt_gt
    jlo0 = jnp.full((blk, 1), -1, jnp.int32)
    jhi0 = jnp.full((blk, 1), n - 1, jnp.int32)

    def jb(_, carry):
      jlo, jhi = carry
      mid = (jlo + jhi) >> 1
      cnt = jnp.sum((eq & (col <= mid)).astype(jnp.int32), axis=1,
                    keepdims=True)
      ok = cnt >= m
      return jnp.where(ok, jlo, mid + 1), jnp.where(ok, mid, jhi)

    _, jstar = lax.fori_loop(0, 13, jb, (jlo0, jhi0))
    return jstar

  jstar = lax.cond(jnp.any(cnt_ge != TOPK), with_tie, no_tie)
  mask = ge & ((s != t_f) | (col <= jstar))

  w = jnp.where(mask, s, 0.0)
  ssum = jnp.maximum(jnp.sum(w, axis=1, keepdims=True), 1e-8)
  token = jnp.dot(w.astype(jnp.bfloat16), feat_ref[...],
                  preferred_element_type=jnp.float32) / ssum
  proj = lax.dot_general(token, w_ref[0], (((1,), (1,)), ((), ())),
                         preferred_element_type=jnp.float32)
  out_ref[0] = jnp.maximum(proj + b_ref[0], 0.0)


def _topk_agg_call(sim, feat, proj_w, proj_b):
  v, n, _ = sim.shape
  d = feat.shape[1]
  blk = min(_ROW_BLK, n)
  grid = (v, n // blk)
  return pl.pallas_call(
      _topk_agg_body,
      grid=grid,
      in_specs=[
          pl.BlockSpec((1, blk, n), lambda i, j: (i, j, 0)),
          pl.BlockSpec((n, d), lambda i, j: (0, 0)),
          pl.BlockSpec((1, d, d), lambda i, j: (i, 0, 0)),
          pl.BlockSpec((1, 1, d), lambda i, j: (i, 0, 0)),
      ],
      out_specs=pl.BlockSpec((1, blk, d), lambda i, j: (i, j, 0)),
      out_shape=jax.ShapeDtypeStruct((v, n, d), jnp.float32),
      compiler_params=pltpu.CompilerParams(
          dimension_semantics=("arbitrary", "arbitrary")),
  )(sim, feat, proj_w, proj_b.reshape(v, 1, d))


def _gather_rows(table, idx):
  nrows, d = table.shape
  b = idx.shape[0]
  per_w = b // _NW
  nch = per_w // _GCHUNK
  mesh = plsc.VectorSubcoreMesh(core_axis_name="c", subcore_axis_name="s",
                                num_cores=_NC, num_subcores=_NS)

  @functools.partial(
      pl.kernel,
      mesh=mesh,
      out_type=jax.ShapeDtypeStruct((b, d), jnp.float32),
      scratch_types=[
          pltpu.VMEM((_GCHUNK,), jnp.int32),
          pltpu.VMEM((_GCHUNK, d), jnp.float32),
          pltpu.SemaphoreType.DMA,
      ],
  )
  def k(idx_hbm, table_hbm, out_hbm, idx_v, rows_v, sem):
    wid = lax.axis_index("s") * _NC + lax.axis_index("c")
    base = wid * per_w
    for c in range(nch):
      off = base + c * _GCHUNK
      pltpu.sync_copy(idx_hbm.at[pl.ds(off, _GCHUNK)], idx_v)
      pltpu.async_copy(table_hbm.at[idx_v], rows_v, sem).wait()
      pltpu.sync_copy(rows_v, out_hbm.at[pl.ds(off, _GCHUNK)])

  return k(idx, table)


def _att_fusion(tok, emb, wq, bq, wk, bk, wv, bv, wo, bo, d, norm):
  t0 = tok[:, :d]
  t1 = tok[:, d:2 * d]
  t2 = tok[:, 2 * d:]

  def lin(x, wmat, bvec):
    return lax.dot_general(x, wmat, (((1,), (1,)), ((), ())),
                           preferred_element_type=jnp.float32) + bvec

  q = lin(emb, wq, bq)
  s0 = jnp.sum(q * lin(t0, wk, bk), axis=1, keepdims=True) * norm
  s1 = jnp.sum(q * lin(t1, wk, bk), axis=1, keepdims=True) * norm
  s2 = jnp.sum(q * lin(t2, wk, bk), axis=1, keepdims=True) * norm
  mx = jnp.maximum(jnp.maximum(s0, s1), s2)
  e0 = jnp.exp(s0 - mx)
  e1 = jnp.exp(s1 - mx)
  e2 = jnp.exp(s2 - mx)
  z = e0 + e1 + e2
  fused = (e0 * lin(t0, wv, bv) + e1 * lin(t1, wv, bv)
           + e2 * lin(t2, wv, bv)) / z
  mean = (t0 + t1 + t2) * (1.0 / 3.0)
  var = ((t0 - mean) ** 2 + (t1 - mean) ** 2 + (t2 - mean) ** 2) * 0.5
  dis = jnp.sqrt(var)
  cat = jnp.concatenate((fused, dis), axis=1)
  return jnp.maximum(lin(cat, wo, bo), 0.0)


def _fusion_mlp_body(mtok_ref, dtok_ref, memb_ref, demb_ref, conf_ref,
                     mWq, mbq, mWk, mbk, mWv, mbv, mWo, mbo,
                     dWq, dbq, dWk, dbk, dWv, dbv, dWo, dbo,
                     W1a, w1c, b1, W2, b2, out_ref):
  d = memb_ref.shape[1]
  norm = 1.0 / (d ** 0.5)
  mi_emb = memb_ref[...]
  dj_emb = demb_ref[...]
  mi_sem = _att_fusion(mtok_ref[...], mi_emb, mWq[...], mbq[...], mWk[...],
                       mbk[...], mWv[...], mbv[...], mWo[...], mbo[...], d,
                       norm)
  dj_sem = _att_fusion(dtok_ref[...], dj_emb, dWq[...], dbq[...], dWk[...],
                       dbk[...], dWv[...], dbv[...], dWo[...], dbo[...], d,
                       norm)
  big = jnp.concatenate(
      (mi_emb, dj_emb, mi_sem, dj_sem, jnp.abs(mi_sem - dj_sem),
       mi_sem * dj_sem), axis=1)
  h = lax.dot_general(big, W1a[...], (((1,), (1,)), ((), ())),
                      preferred_element_type=jnp.float32)
  h = jnp.maximum(h + conf_ref[...] * w1c[...] + b1[...], 0.0)
  out = lax.dot_general(h, W2[...], (((1,), (1,)), ((), ())),
                        preferred_element_type=jnp.float32)
  out_ref[...] = jnp.maximum(out + b2[...], 0.0)


def _fusion_mlp_call(m_tok, d_tok, mi_emb, dj_emb, conf, mf, df, W1a, w1c,
                     b1, W2, b2):
  b, d = mi_emb.shape
  hid = W2.shape[0]
  blk = min(_PAIR_BLK, b)
  grid = (b // blk,)

  def rowblk(shape):
    return pl.BlockSpec((blk,) + shape[1:], lambda i: (i,) + (0,) * (len(shape) - 1))

  def whole(shape):
    return pl.BlockSpec(shape, lambda i: (0,) * len(shape))

  fspecs = []
  fargs = []
  for side in (mf, df):
    for arr in side:
      a = arr if arr.ndim == 2 else arr.reshape(1, -1)
      fargs.append(a)
      fspecs.append(whole(a.shape))

  args = [m_tok, d_tok, mi_emb, dj_emb, conf.reshape(b, 1)] + fargs + [
      W1a, w1c.reshape(1, hid), b1.reshape(1, hid), W2, b2.reshape(1, hid)]
  specs = [rowblk(m_tok.shape), rowblk(d_tok.shape), rowblk(mi_emb.shape),
           rowblk(dj_emb.shape), rowblk((b, 1))] + fspecs + [
      whole(W1a.shape), whole((1, hid)), whole((1, hid)), whole(W2.shape),
      whole((1, hid))]

  return pl.pallas_call(
      _fusion_mlp_body,
      grid=grid,
      in_specs=specs,
      out_specs=rowblk((b, hid)),
      out_shape=jax.ShapeDtypeStruct((b, hid), jnp.float32),
      compiler_params=pltpu.CompilerParams(
          dimension_semantics=("arbitrary",)),
  )(*args)


def kernel(sim_mm, sim_dd, all_node_feat, m_node, d_node, mi_emb, dj_emb,
           pair_confidence, m_proj_W, m_proj_b, d_proj_W, d_proj_b,
           mf_Wq, mf_bq, mf_Wk, mf_bk, mf_Wv, mf_bv, mf_Wo, mf_bo,
           df_Wq, df_bq, df_Wk, df_bk, df_Wv, df_bv, df_Wo, df_bo,
           pm_W1, pm_b1, pm_W2, pm_b2):
  v, n_m, _ = sim_mm.shape
  n_d = sim_dd.shape[1]
  d = all_node_feat.shape[1]
  b = m_node.shape[0]

  mi_feat = all_node_feat[:n_m]
  di_feat = all_node_feat[n_m:]

  proj_m = _topk_agg_call(sim_mm, mi_feat.astype(jnp.bfloat16), m_proj_W,
                          m_proj_b)
  proj_d = _topk_agg_call(sim_dd, di_feat.astype(jnp.bfloat16), d_proj_W,
                          d_proj_b)

  offs_m = jnp.arange(v, dtype=jnp.int32) * n_m
  offs_d = jnp.arange(v, dtype=jnp.int32) * n_d
  idx_m = (m_node.astype(jnp.int32)[:, None] + offs_m[None, :]).reshape(-1)
  idx_d = (d_node.astype(jnp.int32)[:, None] + offs_d[None, :]).reshape(-1)

  m_tok = _gather_rows(proj_m.reshape(v * n_m, d), idx_m).reshape(b, v * d)
  d_tok = _gather_rows(proj_d.reshape(v * n_d, d), idx_d).reshape(b, v * d)

  mf = (mf_Wq, mf_bq, mf_Wk, mf_bk, mf_Wv, mf_bv, mf_Wo, mf_bo)
  df = (df_Wq, df_bq, df_Wk, df_bk, df_Wv, df_bv, df_Wo, df_bo)
  d_in = pm_W1.shape[1] - 1
  return _fusion_mlp_call(m_tok, d_tok, mi_emb, dj_emb, pair_confidence,
                          mf, df, pm_W1[:, :d_in], pm_W1[:, d_in],
                          pm_b1, pm_W2, pm_b2)


# speedup vs baseline: 1.1714x; 1.0056x over previous
import functools

import jax
import jax.numpy as jnp
from jax import lax
from jax.experimental import pallas as pl
from jax.experimental.pallas import tpu as pltpu
from jax.experimental.pallas import tpu_sc as plsc

TOPK = 32
_ROW_BLK = 512
_PAIR_BLK = 512

_NC = 2
_NS = 16
_NW = _NC * _NS
_GCHUNK = 64


def _topk_agg_body(sim_ref, feat_ref, w_ref, b_ref, out_ref):
  blk, n = sim_ref.shape[1], sim_ref.shape[2]
  j = pl.program_id(1)
  s = sim_ref[0]
  col = lax.broadcasted_iota(jnp.int32, (blk, n), 1)
  row = lax.broadcasted_iota(jnp.int32, (blk, n), 0) + j * blk
  s = jnp.where(col == row, 0.0, s)

  nch = n // 128
  m1 = s[:, :128]
  for c in range(1, nch):
    m1 = jnp.maximum(m1, s[:, c * 128:(c + 1) * 128])
  m2 = jnp.maximum(jnp.maximum(m1[:, :32], m1[:, 32:64]),
                   jnp.maximum(m1[:, 64:96], m1[:, 96:128]))
  lo0 = lax.bitcast_convert_type(jnp.min(m2, axis=1, keepdims=True),
                                 jnp.int32)
  hi0 = lax.bitcast_convert_type(jnp.max(m2, axis=1, keepdims=True),
                                 jnp.int32)

  def bis_cond(carry):
    i, lo, hi = carry
    return jnp.logical_and(i < 32, jnp.any(lo < hi))

  def bis(carry):
    i, lo, hi = carry
    for _ in range(4):
      mid = lo + ((hi - lo + 1) >> 1)
      mid_f = lax.bitcast_convert_type(mid, jnp.float32)
      cnt = jnp.sum((s >= mid_f).astype(jnp.int32), axis=1, keepdims=True)
      ge = cnt >= TOPK
      lo = jnp.where(ge, mid, lo)
      hi = jnp.where(ge, hi, mid - 1)
    return i + 4, lo, hi

  _, t, _ = lax.while_loop(bis_cond, bis, (jnp.int32(0), lo0, hi0))

  t_f = lax.bitcast_convert_type(t, jnp.float32)
  ge = s >= t_f
  cnt_ge = jnp.sum(ge.astype(jnp.int32), axis=1, keepdims=True)

  def no_tie():
    return jnp.full((blk, 1), n - 1, jnp.int32)

  def with_tie():
    eq = ge & (s == t_f)
    cnt_gt = cnt_ge - jnp.sum(eq.astype(jnp.int32), axis=1, keepdims=True)
    m = TOPK - cnt_gt
    jlo0 = jnp.full((blk, 1), -1, jnp.int32)
    jhi0 = jnp.full((blk, 1), n - 1, jnp.int32)

    def jb(_, carry):
      jlo, jhi = carry
      mid = (jlo + jhi) >> 1
      cnt = jnp.sum((eq & (col <= mid)).astype(jnp.int32), axis=1,
                    keepdims=True)
      ok = cnt >= m
      return jnp.where(ok, jlo, mid + 1), jnp.where(ok, mid, jhi)

    _, jstar = lax.fori_loop(0, 13, jb, (jlo0, jhi0))
    return jstar

  jstar = lax.cond(jnp.any(cnt_ge != TOPK), with_tie, no_tie)
  mask = ge & ((s != t_f) | (col <= jstar))

  w = jnp.where(mask, s, 0.0)
  ssum = jnp.maximum(jnp.sum(w, axis=1, keepdims=True), 1e-8)
  token = jnp.dot(w.astype(jnp.bfloat16), feat_ref[...],
                  preferred_element_type=jnp.float32) / ssum
  proj = lax.dot_general(token, w_ref[0], (((1,), (1,)), ((), ())),
                         preferred_element_type=jnp.float32)
  out_ref[0] = jnp.maximum(proj + b_ref[0], 0.0)


def _topk_agg_call(sim, feat, proj_w, proj_b):
  v, n, _ = sim.shape
  d = feat.shape[1]
  blk = min(_ROW_BLK, n)
  grid = (v, n // blk)
  return pl.pallas_call(
      _topk_agg_body,
      grid=grid,
      in_specs=[
          pl.BlockSpec((1, blk, n), lambda i, j: (i, j, 0)),
          pl.BlockSpec((n, d), lambda i, j: (0, 0)),
          pl.BlockSpec((1, d, d), lambda i, j: (i, 0, 0)),
          pl.BlockSpec((1, 1, d), lambda i, j: (i, 0, 0)),
      ],
      out_specs=pl.BlockSpec((1, blk, d), lambda i, j: (i, j, 0)),
      out_shape=jax.ShapeDtypeStruct((v, n, d), jnp.float32),
      compiler_params=pltpu.CompilerParams(
          dimension_semantics=("arbitrary", "arbitrary")),
  )(sim, feat, proj_w, proj_b.reshape(v, 1, d))


def _gather_tokens(table_m, idx_m, table_d, idx_d):
  d = table_m.shape[1]
  b = idx_m.shape[0]
  half = _NW // 2
  per_w = b // half
  nch = per_w // _GCHUNK
  mesh = plsc.VectorSubcoreMesh(core_axis_name="c", subcore_axis_name="s",
                                num_cores=_NC, num_subcores=_NS)

  im = idx_m.reshape(half, nch, _GCHUNK)
  iD = idx_d.reshape(half, nch, _GCHUNK)

  @functools.partial(
      pl.kernel,
      mesh=mesh,
      out_type=jax.ShapeDtypeStruct((2 * b, d), jnp.float32),
      scratch_types=[
          pltpu.VMEM((nch, _GCHUNK), jnp.int32),
          pltpu.VMEM((2, _GCHUNK, d), jnp.float32),
          pltpu.SemaphoreType.DMA,
          pltpu.SemaphoreType.DMA,
      ],
  )
  def k(im_hbm, tm_hbm, id_hbm, td_hbm, out_hbm, idx_v, rows_v, sem0, sem1):
    wid = lax.axis_index("s") * _NC + lax.axis_index("c")
    sems = (sem0, sem1)

    def do_side(idx_hbm, table_hbm, out_off, lwid):
      base = out_off + lwid * per_w
      pltpu.sync_copy(idx_hbm.at[lwid], idx_v)

      def start(c):
        return pltpu.async_copy(table_hbm.at[idx_v.at[c]],
                                rows_v.at[c % 2], sems[c % 2])

      desc = {0: start(0)}
      for c in range(nch):
        if c + 1 < nch:
          desc[c + 1] = start(c + 1)
        desc[c].wait()
        pltpu.sync_copy(rows_v.at[c % 2],
                        out_hbm.at[pl.ds(base + c * _GCHUNK, _GCHUNK)])

    @pl.when(wid < half)
    def _():
      do_side(im_hbm, tm_hbm, 0, wid)

    @pl.when(wid >= half)
    def _():
      do_side(id_hbm, td_hbm, b, wid - half)

  return k(im, table_m, iD, table_d)


def _att_fusion(tok, emb, wq, bq, wk, bk, wv, bv, wo, bo, d, norm):
  t0 = tok[:, :d]
  t1 = tok[:, d:2 * d]
  t2 = tok[:, 2 * d:]

  def lin(x, wmat, bvec):
    return lax.dot_general(x, wmat, (((1,), (1,)), ((), ())),
                           preferred_element_type=jnp.float32) + bvec

  q = lin(emb, wq, bq)
  s0 = jnp.sum(q * lin(t0, wk, bk), axis=1, keepdims=True) * norm
  s1 = jnp.sum(q * lin(t1, wk, bk), axis=1, keepdims=True) * norm
  s2 = jnp.sum(q * lin(t2, wk, bk), axis=1, keepdims=True) * norm
  mx = jnp.maximum(jnp.maximum(s0, s1), s2)
  e0 = jnp.exp(s0 - mx)
  e1 = jnp.exp(s1 - mx)
  e2 = jnp.exp(s2 - mx)
  z = e0 + e1 + e2
  fused = (e0 * lin(t0, wv, bv) + e1 * lin(t1, wv, bv)
           + e2 * lin(t2, wv, bv)) / z
  mean = (t0 + t1 + t2) * (1.0 / 3.0)
  var = ((t0 - mean) ** 2 + (t1 - mean) ** 2 + (t2 - mean) ** 2) * 0.5
  dis = jnp.sqrt(var)
  cat = jnp.concatenate((fused, dis), axis=1)
  return jnp.maximum(lin(cat, wo, bo), 0.0)


def _fusion_mlp_body(mtok_ref, dtok_ref, memb_ref, demb_ref, conf_ref,
                     mWq, mbq, mWk, mbk, mWv, mbv, mWo, mbo,
                     dWq, dbq, dWk, dbk, dWv, dbv, dWo, dbo,
                     W1a, w1c, b1, W2, b2, out_ref):
  d = memb_ref.shape[1]
  norm = 1.0 / (d ** 0.5)
  mi_emb = memb_ref[...]
  dj_emb = demb_ref[...]
  mi_sem = _att_fusion(mtok_ref[...], mi_emb, mWq[...], mbq[...], mWk[...],
                       mbk[...], mWv[...], mbv[...], mWo[...], mbo[...], d,
                       norm)
  dj_sem = _att_fusion(dtok_ref[...], dj_emb, dWq[...], dbq[...], dWk[...],
                       dbk[...], dWv[...], dbv[...], dWo[...], dbo[...], d,
                       norm)
  big = jnp.concatenate(
      (mi_emb, dj_emb, mi_sem, dj_sem, jnp.abs(mi_sem - dj_sem),
       mi_sem * dj_sem), axis=1)
  h = lax.dot_general(big.astype(jnp.bfloat16), W1a[...],
                      (((1,), (1,)), ((), ())),
                      preferred_element_type=jnp.float32)
  h = jnp.maximum(h + conf_ref[...] * w1c[...] + b1[...], 0.0)
  out = lax.dot_general(h, W2[...], (((1,), (1,)), ((), ())),
                        preferred_element_type=jnp.float32)
  out_ref[...] = jnp.maximum(out + b2[...], 0.0)


def _fusion_mlp_call(m_tok, d_tok, mi_emb, dj_emb, conf, mf, df, W1a, w1c,
                     b1, W2, b2):
  b, d = mi_emb.shape
  hid = W2.shape[0]
  blk = min(_PAIR_BLK, b)
  grid = (b // blk,)

  def rowblk(shape):
    return pl.BlockSpec((blk,) + shape[1:], lambda i: (i,) + (0,) * (len(shape) - 1))

  def whole(shape):
    return pl.BlockSpec(shape, lambda i: (0,) * len(shape))

  fspecs = []
  fargs = []
  for side in (mf, df):
    for arr in side:
      a = arr if arr.ndim == 2 else arr.reshape(1, -1)
      fargs.append(a)
      fspecs.append(whole(a.shape))

  args = [m_tok, d_tok, mi_emb, dj_emb, conf.reshape(b, 1)] + fargs + [
      W1a, w1c.reshape(1, hid), b1.reshape(1, hid), W2, b2.reshape(1, hid)]
  specs = [rowblk(m_tok.shape), rowblk(d_tok.shape), rowblk(mi_emb.shape),
           rowblk(dj_emb.shape), rowblk((b, 1))] + fspecs + [
      whole(W1a.shape), whole((1, hid)), whole((1, hid)), whole(W2.shape),
      whole((1, hid))]

  return pl.pallas_call(
      _fusion_mlp_body,
      grid=grid,
      in_specs=specs,
      out_specs=rowblk((b, hid)),
      out_shape=jax.ShapeDtypeStruct((b, hid), jnp.float32),
      compiler_params=pltpu.CompilerParams(
          dimension_semantics=("arbitrary",)),
  )(*args)


def kernel(sim_mm, sim_dd, all_node_feat, m_node, d_node, mi_emb, dj_emb,
           pair_confidence, m_proj_W, m_proj_b, d_proj_W, d_proj_b,
           mf_Wq, mf_bq, mf_Wk, mf_bk, mf_Wv, mf_bv, mf_Wo, mf_bo,
           df_Wq, df_bq, df_Wk, df_bk, df_Wv, df_bv, df_Wo, df_bo,
           pm_W1, pm_b1, pm_W2, pm_b2):
  v, n_m, _ = sim_mm.shape
  n_d = sim_dd.shape[1]
  d = all_node_feat.shape[1]
  b = m_node.shape[0]

  mi_feat = all_node_feat[:n_m]
  di_feat = all_node_feat[n_m:]

  proj_m = _topk_agg_call(sim_mm, mi_feat.astype(jnp.bfloat16), m_proj_W,
                          m_proj_b)
  proj_d = _topk_agg_call(sim_dd, di_feat.astype(jnp.bfloat16), d_proj_W,
                          d_proj_b)

  offs_m = jnp.arange(v, dtype=jnp.int32) * n_m
  offs_d = jnp.arange(v, dtype=jnp.int32) * n_d
  idx_m = (m_node.astype(jnp.int32)[:, None] + offs_m[None, :]).reshape(-1)
  idx_d = (d_node.astype(jnp.int32)[:, None] + offs_d[None, :]).reshape(-1)

  toks = _gather_tokens(proj_m.reshape(v * n_m, d), idx_m,
                        proj_d.reshape(v * n_d, d), idx_d)
  m_tok = toks[:b * v].reshape(b, v * d)
  d_tok = toks[b * v:].reshape(b, v * d)

  mf = (mf_Wq, mf_bq, mf_Wk, mf_bk, mf_Wv, mf_bv, mf_Wo, mf_bo)
  df = (df_Wq, df_bq, df_Wk, df_bk, df_Wv, df_bv, df_Wo, df_bo)
  d_in = pm_W1.shape[1] - 1
  return _fusion_mlp_call(m_tok, d_tok, mi_emb, dj_emb, pair_confidence,
                          mf, df, pm_W1[:, :d_in].astype(jnp.bfloat16),
                          pm_W1[:, d_in], pm_b1, pm_W2, pm_b2)
